# baseline (speedup 1.0000x reference)
import jax
import jax.numpy as jnp
from jax import lax
from jax.experimental import pallas as pl
from jax.experimental.pallas import tpu as pltpu
from jax.experimental.pallas import tpu_sc as plsc

D = 64
DP = D // 2
LANES = 16
VPT = D // LANES
BOM_ID = 3
NC, NS = 2, 16
NW = NC * NS
CHUNK = 128
TOK_PER_WORKER = 819200 // NW
NCHUNK = TOK_PER_WORKER // CHUNK

NGATHER = 14
G_ZROW = (16, 100000, 100000, 1000, 1000, 1000, 1000, 1000, 100000,
          100000, 1000, 1000, 100000, 100000)
STREAM_G = (1, 2, 8, 9, 12, 13)
STREAM_TAB = (0, 0, 1, 2, 2, 2)
BF_FIELDS = (3, 4, 5, 6, 7, 10, 11)
BF_TAB = (0, 0, 0, 0, 0, 1, 2)


def _body(*refs):
    idx_hbm = refs[0:NGATHER]
    type_tab = refs[NGATHER]
    stabs = refs[NGATHER + 1:NGATHER + 4]
    btabs_hbm = refs[NGATHER + 4:NGATHER + 7]
    out = refs[NGATHER + 7]
    it = iter(refs[NGATHER + 8:])
    idxb = [next(it) for _ in range(NGATHER)]
    rbufs = [next(it) for _ in range(2)]
    acc = next(it)
    ltype = next(it)
    lbf = [next(it) for _ in range(3)]
    rsems = [next(it) for _ in range(2)]
    semi = next(it)

    cid = lax.axis_index("c")
    sid = lax.axis_index("s")
    wid = sid * NC + cid
    base_row = wid * NCHUNK

    pltpu.sync_copy(type_tab, ltype)
    for j in range(3):
        pltpu.sync_copy(btabs_hbm[j], lbf[j])

    def accumulate(buf, first):
        if first:
            @plsc.parallel_loop(0, CHUNK, unroll=2)
            def _(t, buf=buf):
                for j in range(VPT):
                    jl = pl.ds(j * LANES, LANES)
                    acc[t, jl] = buf[t, jl]
        else:
            @plsc.parallel_loop(0, CHUNK, unroll=2)
            def _(t, buf=buf):
                for j in range(VPT):
                    jl = pl.ds(j * LANES, LANES)
                    plsc.addupdate(acc.at[t, jl], buf[t, jl])

    def chunk(c, _):
        r0 = base_row + c
        icps = [
            pltpu.async_copy(idx_hbm[g].at[pl.ds(r0, 1)], idxb[g], semi)
            for g in range(NGATHER)
        ]
        for cp in icps:
            cp.wait()
        for i in range(CHUNK // LANES):
            sl = (0, pl.ds(i * LANES, LANES))
            tv = idxb[0][sl]
            m = tv == BOM_ID
            for g in range(1, NGATHER):
                zk = jnp.full((LANES,), G_ZROW[g], jnp.int32)
                iv = idxb[g][sl]
                if g >= 12:
                    idxb[g][sl] = jnp.where(m, iv, zk)
                else:
                    idxb[g][sl] = jnp.where(m, zk, iv)
            idxb[0][sl] = jnp.where(
                m, jnp.full((LANES,), G_ZROW[0], jnp.int32), tv)

        def fire(k):
            return pltpu.async_copy(
                stabs[STREAM_TAB[k]].at[idxb[STREAM_G[k]].at[0]],
                rbufs[k % 2], rsems[k % 2])

        cps = [fire(0), fire(1)]

        @plsc.parallel_loop(0, CHUNK // LANES, unroll=1)
        def _vld(q):
            qsl = (0, pl.ds(q * LANES, LANES))
            tok_v = lax.iota(jnp.int32, LANES) + q * LANES
            rows = [idxb[g][qsl] for g in BF_FIELDS]
            trow = idxb[0][qsl]
            for p in range(DP):
                pv = jnp.full((LANES,), p, jnp.int32)
                se = plsc.load_gather(ltype, [trow, 2 * pv])
                so = plsc.load_gather(ltype, [trow, 2 * pv + 1])
                for f in range(len(BF_FIELDS)):
                    w = plsc.load_gather(lbf[BF_TAB[f]], [rows[f], pv])
                    e, o = plsc.unpack(plsc.bitcast(w, jnp.bfloat16),
                                       format=plsc.PackFormat.INTERLEAVED)
                    se = se + e
                    so = so + o
                plsc.store_scatter(acc, [tok_v, 2 * pv], se)
                plsc.store_scatter(acc, [tok_v, 2 * pv + 1], so)

        for k in range(len(STREAM_G)):
            cps[k % 2].wait()
            accumulate(rbufs[k % 2], first=False)
            if k + 2 < len(STREAM_G):
                cps[k % 2] = fire(k + 2)
        pltpu.sync_copy(acc, out.at[pl.ds(r0 * CHUNK, CHUNK), :])
        return ()

    lax.fori_loop(0, NCHUNK, chunk, ())


def kernel(type, location, source_location, start_time, end_time,
           request_time, commit_time, lead_time, demand, material, method,
           quantity, parent, child, W_type, W_loc, W_time, W_demand, W_mat,
           W_method, W_qty):
    b, l = type.shape
    n = b * l
    idx_arrays = (type, location, source_location, start_time, end_time,
                  request_time, commit_time, lead_time, demand, material,
                  method, quantity, parent, child)
    idxs = [x.reshape(n // CHUNK, CHUNK) for x in idx_arrays]

    def zrow(w):
        return jnp.concatenate([w, jnp.zeros((1, D), w.dtype)], axis=0)

    def bfpack(w):
        wz = zrow(w).astype(jnp.bfloat16)
        v = wz.shape[0]
        return lax.bitcast_convert_type(
            wz.reshape(v, DP, 2), jnp.int32)

    type_z = zrow(W_type)
    stream_tabs = [zrow(W_loc), zrow(W_demand), zrow(W_mat)]
    packed = [bfpack(W_time), bfpack(W_method), bfpack(W_qty)]

    mesh = plsc.VectorSubcoreMesh(core_axis_name="c", subcore_axis_name="s")
    scratch = (
        [pltpu.VMEM((1, CHUNK), jnp.int32) for _ in range(NGATHER)]
        + [pltpu.VMEM((CHUNK, D), jnp.float32) for _ in range(2)]
        + [pltpu.VMEM((CHUNK, D), jnp.float32)]
        + [pltpu.VMEM((17, D), jnp.float32)]
        + [pltpu.VMEM((1001, DP), jnp.int32) for _ in range(3)]
        + [pltpu.SemaphoreType.DMA for _ in range(3)]
    )
    out = pl.kernel(
        _body,
        out_type=jax.ShapeDtypeStruct((n, D), jnp.float32),
        mesh=mesh,
        scratch_types=scratch,
        compiler_params=pltpu.CompilerParams(
            use_tc_tiling_on_sc=False, needs_layout_passes=False),
    )(*idxs, type_z, *stream_tabs, *packed)
    return out.reshape(b, l, D)
